# baseline (speedup 1.0000x reference)
import functools

import jax
import jax.numpy as jnp
from jax import lax
from jax.experimental import pallas as pl
from jax.experimental.pallas import tpu as pltpu
from jax.experimental.pallas import tpu_sc as plsc

NUM_USERS = 100000
EMBED_DIM = 64
BATCH = 4096

NC = 2
NS = 16
L = 16
NW = NC * NS
B_PER_W = BATCH // NW
D_CHUNKS = EMBED_DIM // L
WIDE = 2 * EMBED_DIM


def _tae_kernel(users_hbm, ts_hbm, table_hbm, w_hbm, b_hbm, out_hbm,
                idx_v, tid_v, t_v, pairs_v, out_v, w_v, bias_v, sem):
    wid = lax.axis_index("s") * NC + lax.axis_index("c")
    base = wid * B_PER_W

    pltpu.sync_copy(users_hbm.at[pl.ds(base, B_PER_W)], idx_v)
    pltpu.sync_copy(ts_hbm.at[pl.ds(base, B_PER_W)], t_v)
    pltpu.sync_copy(w_hbm, w_v)
    pltpu.sync_copy(b_hbm, bias_v)

    def tid_body(g, carry):
        sl = pl.ds(g * L, L)
        tid_v[sl] = lax.shift_right_logical(idx_v[sl], 1)
        return carry
    lax.fori_loop(0, B_PER_W // L, tid_body, 0)

    pltpu.async_copy(table_hbm.at[tid_v], pairs_v, sem).wait()

    w_chunks = [w_v[pl.ds(c * L, L)] for c in range(D_CHUNKS)]
    bias_chunks = [bias_v[pl.ds(c * L, L)] for c in range(D_CHUNKS)]

    def body(g, carry):
        t_chunk = t_v[pl.ds(g * L, L)]
        off_chunk = (idx_v[pl.ds(g * L, L)] & 1) * EMBED_DIM
        for j in range(L):
            b = g * L + j
            off = off_chunk[j]
            tb = jnp.full((L,), t_chunk[j])
            for c in range(D_CHUNKS):
                out_v[b, pl.ds(c * L, L)] = (
                    pairs_v[b, pl.ds(off + c * L, L)]
                    + tb * w_chunks[c] + bias_chunks[c])
        return carry
    lax.fori_loop(0, B_PER_W // L, body, 0)

    pltpu.sync_copy(out_v, out_hbm.at[pl.ds(base, B_PER_W)])


@jax.jit
def _run(users, timestamps, table2, w_flat, time_b):
    mesh = plsc.VectorSubcoreMesh(core_axis_name="c", subcore_axis_name="s",
                                  num_cores=NC)
    return pl.kernel(
        _tae_kernel,
        out_type=jax.ShapeDtypeStruct((BATCH, EMBED_DIM), jnp.float32),
        mesh=mesh,
        scratch_types=[
            pltpu.VMEM((B_PER_W,), jnp.int32),
            pltpu.VMEM((B_PER_W,), jnp.int32),
            pltpu.VMEM((B_PER_W,), jnp.float32),
            pltpu.VMEM((B_PER_W, WIDE), jnp.float32),
            pltpu.VMEM((B_PER_W, EMBED_DIM), jnp.float32),
            pltpu.VMEM((EMBED_DIM,), jnp.float32),
            pltpu.VMEM((EMBED_DIM,), jnp.float32),
            pltpu.SemaphoreType.DMA,
        ],
    )(users, timestamps, table2, w_flat, time_b)


def kernel(users, timestamps, table, time_w, time_b):
    table2 = table.reshape(NUM_USERS // 2, WIDE)
    return _run(users.astype(jnp.int32), timestamps, table2,
                time_w.reshape(EMBED_DIM), time_b)


# speedup vs baseline: 1.3004x; 1.3004x over previous
import functools

import jax
import jax.numpy as jnp
from jax import lax
from jax.experimental import pallas as pl
from jax.experimental.pallas import tpu as pltpu
from jax.experimental.pallas import tpu_sc as plsc

NUM_USERS = 100000
EMBED_DIM = 64
BATCH = 4096

NC = 2
NS = 16
L = 16
NW = NC * NS
B_PER_W = BATCH // NW
N_GROUPS = B_PER_W // L


def _tae_kernel(users_hbm, ts_hbm, table_hbm, w_hbm, b_hbm, out_hbm,
                idx_v, t_v, rows_v, w_v, bias_v, sem):
    wid = lax.axis_index("s") * NC + lax.axis_index("c")
    base = wid * B_PER_W

    pltpu.sync_copy(users_hbm.at[pl.ds(base, B_PER_W)], idx_v)
    pltpu.sync_copy(ts_hbm.at[pl.ds(base, B_PER_W)], t_v)
    pltpu.sync_copy(w_hbm, w_v)
    pltpu.sync_copy(b_hbm, bias_v)

    for d in range(EMBED_DIM):
        pltpu.async_copy(table_hbm.at[d].at[idx_v], rows_v.at[d], sem)

    pltpu.make_async_copy(table_hbm.at[:, pl.ds(0, B_PER_W)], rows_v,
                          sem).wait()

    w_chunks = [w_v[pl.ds(c * L, L)] for c in range(EMBED_DIM // L)]
    bias_chunks = [bias_v[pl.ds(c * L, L)] for c in range(EMBED_DIM // L)]

    for d in range(EMBED_DIM):
        wd = jnp.full((L,), w_chunks[d // L][d % L])
        bd = jnp.full((L,), bias_chunks[d // L][d % L])

        def qbody(q, carry, d=d, wd=wd, bd=bd):
            sl = pl.ds(q * L, L)
            rows_v[d, sl] = rows_v[d, sl] + (t_v[sl] * wd + bd)
            return carry
        lax.fori_loop(0, N_GROUPS, qbody, 0)

    pltpu.sync_copy(rows_v, out_hbm.at[:, pl.ds(base, B_PER_W)])


@jax.jit
def _run(users, timestamps, table_t, w_flat, time_b):
    mesh = plsc.VectorSubcoreMesh(core_axis_name="c", subcore_axis_name="s",
                                  num_cores=NC)
    return pl.kernel(
        _tae_kernel,
        out_type=jax.ShapeDtypeStruct((EMBED_DIM, BATCH), jnp.float32),
        mesh=mesh,
        compiler_params=pltpu.CompilerParams(use_tc_tiling_on_sc=False),
        scratch_types=[
            pltpu.VMEM((B_PER_W,), jnp.int32),
            pltpu.VMEM((B_PER_W,), jnp.float32),
            pltpu.VMEM((EMBED_DIM, B_PER_W), jnp.float32),
            pltpu.VMEM((EMBED_DIM,), jnp.float32),
            pltpu.VMEM((EMBED_DIM,), jnp.float32),
            pltpu.SemaphoreType.DMA,
        ],
    )(users, timestamps, table_t, w_flat, time_b)


def kernel(users, timestamps, table, time_w, time_b):
    out_t = _run(users.astype(jnp.int32), timestamps, table.T,
                 time_w.reshape(EMBED_DIM), time_b)
    return out_t.T


# speedup vs baseline: 1.4181x; 1.0905x over previous
import functools

import jax
import jax.numpy as jnp
from jax import lax
from jax.experimental import pallas as pl
from jax.experimental.pallas import tpu as pltpu
from jax.experimental.pallas import tpu_sc as plsc

NUM_USERS = 100000
EMBED_DIM = 64
BATCH = 4096

NC = 2
NS = 16
L = 16
NW = NC * NS
UT_FULL = NUM_USERS // 128
UT_TAIL_BASE = UT_FULL * 128
TAIL_N = NUM_USERS - UT_TAIL_BASE
SUB_UT = 6
SLAB_W = SUB_UT * 128
N_SUB = 5
N_LISTS = N_SUB + 1
OUT_ROWS = BATCH + NW


def _tae_kernel(users_hbm, ts_hbm, table_hbm, tail_hbm, w_hbm, b_hbm,
                out_hbm, users_v, ts_v, slab_v, tail_v, hp_v, buf_v, idx_v,
                w_v, bias_v, sem2):
    wid = lax.axis_index("s") * NC + lax.axis_index("c")
    lo = (UT_FULL * wid) // NW
    hi = (UT_FULL * (wid + 1)) // NW
    trash = BATCH + wid

    pltpu.sync_copy(users_hbm, users_v)
    pltpu.sync_copy(ts_hbm, ts_v)
    pltpu.sync_copy(tail_hbm, tail_v)
    pltpu.sync_copy(w_hbm, w_v)
    pltpu.sync_copy(b_hbm, bias_v)

    iota = lax.iota(jnp.int32, L)
    w_chunks = [w_v[pl.ds(c * L, L)] for c in range(EMBED_DIM // L)]
    bias_chunks = [bias_v[pl.ds(c * L, L)] for c in range(EMBED_DIM // L)]

    subs = []
    for s in range(N_SUB):
        slo = lo + SUB_UT * s
        shi = jnp.minimum(slo + SUB_UT, hi)
        subs.append((s, slo, shi))

    def scan_body(i, curs):
        u16 = users_v[pl.ds(i * L, L)]
        ut16 = lax.shift_right_logical(u16, 7)
        pk = u16 * 4096 + jnp.full((L,), i * L, jnp.int32) + iota
        new = []
        for s, slo, shi in subs:
            m = (ut16 >= slo) & (ut16 < shi)
            plsc.store_compressed(hp_v.at[s, pl.ds(curs[s], L)], pk, mask=m)
            new.append(curs[s] + plsc.all_reduce_population_count(m)[0])
        m = ut16 >= UT_FULL
        plsc.store_compressed(hp_v.at[N_SUB, pl.ds(curs[N_SUB], L)], pk,
                              mask=m)
        new.append(curs[N_SUB] + plsc.all_reduce_population_count(m)[0])
        return tuple(new)
    counts = lax.fori_loop(0, BATCH // L, scan_body,
                           tuple(jnp.int32(0) for _ in range(N_LISTS)))

    def process(list_i, n_hits, width_ref, base_col):
        def chunk_body(k, carry):
            pk = hp_v[list_i, pl.ds(k * L, L)]
            hu = lax.shift_right_logical(pk, 12)
            hb = pk & 4095
            mi = jnp.where(iota + k * L < n_hits, 1, 0).astype(jnp.int32)
            ranks = plsc.cumsum(mi)
            tvals = plsc.load_gather(ts_v, [hb])
            idx_v[pl.ds(0, L)] = jnp.where(mi != 0, hb,
                                           jnp.full((L,), trash))
            for j in range(L):
                @pl.when(mi[j] != 0)
                def _(j=j, hu=hu, ranks=ranks, tvals=tvals):
                    col = hu[j] - base_col
                    row = ranks[j] - 1
                    tb = jnp.full((L,), tvals[j])
                    cid = jnp.full((L,), col)
                    for c in range(EMBED_DIM // L):
                        g = plsc.load_gather(width_ref,
                                             [iota + c * L, cid])
                        buf_v[row, pl.ds(c * L, L)] = (
                            g + tb * w_chunks[c] + bias_chunks[c])
            pltpu.async_copy(buf_v, out_hbm.at[idx_v], sem2).wait()
            return carry
        lax.fori_loop(0, (n_hits + L - 1) // L, chunk_body, 0)

    for s, slo, shi in subs:
        base_ut = jnp.minimum(slo, UT_FULL - SUB_UT)
        pltpu.sync_copy(table_hbm.at[:, pl.ds(base_ut * 128, SLAB_W)],
                        slab_v)
        process(s, counts[s], slab_v, base_ut * 128)

    process(N_SUB, counts[N_SUB], tail_v, jnp.int32(UT_TAIL_BASE))


@jax.jit
def _run(users, timestamps, table_t, tail, w_flat, time_b):
    mesh = plsc.VectorSubcoreMesh(core_axis_name="c", subcore_axis_name="s",
                                  num_cores=NC)
    return pl.kernel(
        _tae_kernel,
        out_type=jax.ShapeDtypeStruct((OUT_ROWS, 128), jnp.float32),
        mesh=mesh,
        compiler_params=pltpu.CompilerParams(needs_layout_passes=False),
        scratch_types=[
            pltpu.VMEM((BATCH,), jnp.int32),
            pltpu.VMEM((BATCH,), jnp.float32),
            pltpu.VMEM((EMBED_DIM, SLAB_W), jnp.float32),
            pltpu.VMEM((EMBED_DIM, TAIL_N), jnp.float32),
            pltpu.VMEM((N_LISTS, BATCH + L), jnp.int32),
            pltpu.VMEM((L, 128), jnp.float32),
            pltpu.VMEM((L,), jnp.int32),
            pltpu.VMEM((EMBED_DIM,), jnp.float32),
            pltpu.VMEM((EMBED_DIM,), jnp.float32),
            pltpu.SemaphoreType.DMA,
        ],
    )(users, timestamps, table_t, tail, w_flat, time_b)


def kernel(users, timestamps, table, time_w, time_b):
    table_t = table.T
    tail = table_t[:, UT_TAIL_BASE:]
    out1 = _run(users.astype(jnp.int32), timestamps, table_t, tail,
                time_w.reshape(EMBED_DIM), time_b)
    return out1[:BATCH, :EMBED_DIM]


# speedup vs baseline: 1.9383x; 1.3668x over previous
import functools

import jax
import jax.numpy as jnp
from jax import lax
from jax.experimental import pallas as pl
from jax.experimental.pallas import tpu as pltpu
from jax.experimental.pallas import tpu_sc as plsc

NUM_USERS = 100000
EMBED_DIM = 64
BATCH = 4096

NC = 2
NS = 16
L = 16
NW = NC * NS
UT_FULL = NUM_USERS // 128
UT_TAIL_BASE = UT_FULL * 128
TAIL_N = NUM_USERS - UT_TAIL_BASE
SUB_UT = 5
SLAB_W = SUB_UT * 128
N_SUB = 5
N_LISTS = N_SUB + 1
OUT_ROWS = BATCH + NW
DC = EMBED_DIM // L


def _tae_kernel(users_hbm, ts_hbm, table_hbm, tail_hbm, w_hbm, b_hbm,
                out_hbm, users_v, ts_v, slab0_v, slab1_v, tail_v,
                hl_v, stg_v, bufa_v, bufb_v, idxa_v, idxb_v, w_v, bias_v,
                sema, semb, sems0, sems1):
    wid = lax.axis_index("s") * NC + lax.axis_index("c")
    lo = (UT_FULL * wid) // NW
    hi = (UT_FULL * (wid + 1)) // NW
    trash16 = jnp.full((L,), BATCH + wid, jnp.int32)

    pltpu.sync_copy(users_hbm, users_v)
    pltpu.sync_copy(ts_hbm, ts_v)
    pltpu.sync_copy(tail_hbm, tail_v)
    pltpu.sync_copy(w_hbm, w_v)
    pltpu.sync_copy(b_hbm, bias_v)

    iota = lax.iota(jnp.int32, L)
    w_chunks = [w_v[pl.ds(c * L, L)] for c in range(DC)]
    bias_chunks = [bias_v[pl.ds(c * L, L)] for c in range(DC)]

    slabs = [slab0_v, slab1_v]
    slab_sems = [sems0, sems1]

    def slab_src(s_lo):
        base_ut = jnp.minimum(s_lo, UT_FULL - SUB_UT)
        return base_ut * 128

    subs = []
    for s in range(N_SUB):
        slo = lo + SUB_UT * s
        subs.append((s, slo, jnp.minimum(slo + SUB_UT, hi)))

    hi_eff = jnp.where(wid == NW - 1, UT_FULL + 1, hi)

    def scan_body(i, cur):
        u16 = users_v[pl.ds(i * L, L)]
        ut16 = lax.shift_right_logical(u16, 7)
        pk = u16 * 4096 + jnp.full((L,), i * L, jnp.int32) + iota
        m = (ut16 >= lo) & (ut16 < hi_eff)
        plsc.store_compressed(hl_v.at[pl.ds(cur, L)], pk, mask=m)
        return cur + plsc.all_reduce_population_count(m)[0]
    nh = lax.fori_loop(0, BATCH // L, scan_body, jnp.int32(0))
    nh_chunks = (nh + L - 1) // L

    def recompact(sub_lo, sub_hi):
        def rbody(k, cur):
            pk = hl_v[pl.ds(k * L, L)]
            ut = lax.shift_right_logical(pk, 12 + 7)
            m = (ut >= sub_lo) & (ut < sub_hi) & (iota + k * L < nh)
            plsc.store_compressed(stg_v.at[pl.ds(cur, L)], pk, mask=m)
            return cur + plsc.all_reduce_population_count(m)[0]
        return lax.fori_loop(0, nh_chunks, rbody, jnp.int32(0))

    pltpu.async_copy(table_hbm.at[:, pl.ds(slab_src(subs[0][1]), SLAB_W)],
                     slabs[0], slab_sems[0])

    def build_chunk(k, n_hits, width_ref, base_col, buf_ref,
                    idx_ref):
        pk = stg_v[pl.ds(k * L, L)]
        hu = lax.shift_right_logical(pk, 12)
        hb = pk & 4095
        p2 = jnp.clip(n_hits - k * L, 0, L)
        idx_ref[pl.ds(0, L)] = jnp.where(iota < p2, hb, trash16)
        tvals = plsc.load_gather(ts_v, [hb])
        base16 = jnp.full((L,), base_col, jnp.int32)

        def hbody(h, carry):
            hvec = jnp.full((L,), h, jnp.int32)
            tb = tvals.at[hvec].get(mode="promise_in_bounds")
            cid = hu.at[hvec].get(mode="promise_in_bounds") - base16
            for c in range(DC):
                g = plsc.load_gather(width_ref, [iota + c * L, cid])
                buf_ref[h, pl.ds(c * L, L)] = (g + tb * w_chunks[c]
                                               + bias_chunks[c])
            return carry
        lax.fori_loop(0, p2, hbody, 0)

    def drain(buf_ref, idx_ref, sem):
        pltpu.make_async_copy(buf_ref, out_hbm.at[idx_ref], sem).wait()

    def fire(buf_ref, idx_ref, sem):
        pltpu.async_copy(buf_ref, out_hbm.at[idx_ref], sem)

    idxa_v[pl.ds(0, L)] = trash16
    idxb_v[pl.ds(0, L)] = trash16
    fire(bufa_v, idxa_v, sema)
    fire(bufb_v, idxb_v, semb)

    def process(n_hits, width_ref, base_col):
        nch2 = (n_hits + 2 * L - 1) // (2 * L)

        def body2(k2, carry):
            drain(bufa_v, idxa_v, sema)
            build_chunk(2 * k2, n_hits, width_ref, base_col,
                        bufa_v, idxa_v)
            fire(bufa_v, idxa_v, sema)
            drain(bufb_v, idxb_v, semb)
            build_chunk(2 * k2 + 1, n_hits, width_ref, base_col,
                        bufb_v, idxb_v)
            fire(bufb_v, idxb_v, semb)
            return carry
        lax.fori_loop(0, nch2, body2, 0)

    for s, slo, shi in subs:
        if s + 1 < N_SUB:
            nxt = subs[s + 1]
            pltpu.async_copy(
                table_hbm.at[:, pl.ds(slab_src(nxt[1]), SLAB_W)],
                slabs[(s + 1) % 2], slab_sems[(s + 1) % 2])
        ns = recompact(slo, shi)
        pltpu.make_async_copy(
            table_hbm.at[:, pl.ds(slab_src(slo), SLAB_W)],
            slabs[s % 2], slab_sems[s % 2]).wait()
        process(ns, slabs[s % 2], slab_src(slo))

    nt = recompact(jnp.int32(UT_FULL), jnp.int32(UT_FULL + 2))
    process(nt, tail_v, jnp.int32(UT_TAIL_BASE))

    drain(bufa_v, idxa_v, sema)
    drain(bufb_v, idxb_v, semb)


@jax.jit
def _run(users, timestamps, table_t, tail, w_flat, time_b):
    mesh = plsc.VectorSubcoreMesh(core_axis_name="c", subcore_axis_name="s",
                                  num_cores=NC)
    return pl.kernel(
        _tae_kernel,
        out_type=jax.ShapeDtypeStruct((OUT_ROWS, 128), jnp.float32),
        mesh=mesh,
        compiler_params=pltpu.CompilerParams(needs_layout_passes=False),
        scratch_types=[
            pltpu.VMEM((BATCH,), jnp.int32),
            pltpu.VMEM((BATCH,), jnp.float32),
            pltpu.VMEM((EMBED_DIM, SLAB_W), jnp.float32),
            pltpu.VMEM((EMBED_DIM, SLAB_W), jnp.float32),
            pltpu.VMEM((EMBED_DIM, TAIL_N), jnp.float32),
            pltpu.VMEM((BATCH + L,), jnp.int32),
            pltpu.VMEM((BATCH + L,), jnp.int32),
            pltpu.VMEM((L, 128), jnp.float32),
            pltpu.VMEM((L, 128), jnp.float32),
            pltpu.VMEM((L,), jnp.int32),
            pltpu.VMEM((L,), jnp.int32),
            pltpu.VMEM((EMBED_DIM,), jnp.float32),
            pltpu.VMEM((EMBED_DIM,), jnp.float32),
            pltpu.SemaphoreType.DMA,
            pltpu.SemaphoreType.DMA,
            pltpu.SemaphoreType.DMA,
            pltpu.SemaphoreType.DMA,
        ],
    )(users, timestamps, table_t, tail, w_flat, time_b)


def kernel(users, timestamps, table, time_w, time_b):
    table_t = table.T
    tail = table_t[:, UT_TAIL_BASE:]
    out1 = _run(users.astype(jnp.int32), timestamps, table_t, tail,
                time_w.reshape(EMBED_DIM), time_b)
    return out1[:BATCH, :EMBED_DIM]


# speedup vs baseline: 2.0049x; 1.0344x over previous
import functools

import jax
import jax.numpy as jnp
from jax import lax
from jax.experimental import pallas as pl
from jax.experimental.pallas import tpu as pltpu
from jax.experimental.pallas import tpu_sc as plsc

NUM_USERS = 100000
EMBED_DIM = 64
BATCH = 4096

NC = 2
NS = 16
L = 16
NW = NC * NS
UT_FULL = NUM_USERS // 128
UT_TAIL_BASE = UT_FULL * 128
TAIL_N = NUM_USERS - UT_TAIL_BASE
SUB_UT = 5
SLAB_W = SUB_UT * 128
N_SUB = 5
N_LISTS = N_SUB + 1
OUT_ROWS = BATCH + NW
DC = EMBED_DIM // L


def _tae_kernel(users_hbm, ts_hbm, table_hbm, tail_hbm, w_hbm, b_hbm,
                out_hbm, users_v, ts_v, slab0_v, slab1_v, tail_v,
                hl_v, stg_v, bufa_v, bufb_v, idxa_v, idxb_v, w_v, bias_v,
                sema, semb, sems0, sems1, semt):
    wid = lax.axis_index("s") * NC + lax.axis_index("c")
    lo = (UT_FULL * wid) // NW
    hi = (UT_FULL * (wid + 1)) // NW
    trash16 = jnp.full((L,), BATCH + wid, jnp.int32)

    iota = lax.iota(jnp.int32, L)
    w_chunks = [w_v[pl.ds(c * L, L)] for c in range(DC)]
    bias_chunks = [bias_v[pl.ds(c * L, L)] for c in range(DC)]

    slabs = [slab0_v, slab1_v]
    slab_sems = [sems0, sems1]

    def slab_src(s_lo):
        base_ut = jnp.minimum(s_lo, UT_FULL - SUB_UT)
        return base_ut * 128

    subs = []
    for s in range(N_SUB):
        slo = lo + SUB_UT * s
        subs.append((s, slo, jnp.minimum(slo + SUB_UT, hi)))

    pltpu.async_copy(table_hbm.at[:, pl.ds(slab_src(subs[0][1]), SLAB_W)],
                     slabs[0], slab_sems[0])
    pltpu.async_copy(table_hbm.at[:, pl.ds(slab_src(subs[1][1]), SLAB_W)],
                     slabs[1], slab_sems[1])
    pltpu.async_copy(tail_hbm, tail_v, semt)
    pltpu.async_copy(users_hbm, users_v, sema)
    pltpu.async_copy(ts_hbm, ts_v, sema)
    pltpu.async_copy(w_hbm, w_v, semb)
    pltpu.async_copy(b_hbm, bias_v, semb)
    pltpu.make_async_copy(users_hbm, users_v, sema).wait()
    pltpu.make_async_copy(ts_hbm, ts_v, sema).wait()
    pltpu.make_async_copy(w_hbm, w_v, semb).wait()
    pltpu.make_async_copy(b_hbm, bias_v, semb).wait()

    hi_eff = jnp.where(wid == NW - 1, UT_FULL + 1, hi)

    def scan_body(i, cur):
        u16 = users_v[pl.ds(i * L, L)]
        ut16 = lax.shift_right_logical(u16, 7)
        pk = u16 * 4096 + jnp.full((L,), i * L, jnp.int32) + iota
        m = (ut16 >= lo) & (ut16 < hi_eff)
        plsc.store_compressed(hl_v.at[pl.ds(cur, L)], pk, mask=m)
        return cur + plsc.all_reduce_population_count(m)[0]
    nh = lax.fori_loop(0, BATCH // L, scan_body, jnp.int32(0))
    nh_chunks = (nh + L - 1) // L

    def recompact(sub_lo, sub_hi):
        def rbody(k, cur):
            pk = hl_v[pl.ds(k * L, L)]
            ut = lax.shift_right_logical(pk, 12 + 7)
            m = (ut >= sub_lo) & (ut < sub_hi) & (iota + k * L < nh)
            plsc.store_compressed(stg_v.at[pl.ds(cur, L)], pk, mask=m)
            return cur + plsc.all_reduce_population_count(m)[0]
        return lax.fori_loop(0, nh_chunks, rbody, jnp.int32(0))

    def build_chunk(k, n_hits, width_ref, base_col, buf_ref,
                    idx_ref):
        pk = stg_v[pl.ds(k * L, L)]
        hu = lax.shift_right_logical(pk, 12)
        hb = pk & 4095
        p2 = jnp.clip(n_hits - k * L, 0, L)
        idx_ref[pl.ds(0, L)] = jnp.where(iota < p2, hb, trash16)
        tvals = plsc.load_gather(ts_v, [hb])
        base16 = jnp.full((L,), base_col, jnp.int32)

        def hbody(h, carry):
            hvec = jnp.full((L,), h, jnp.int32)
            tb = tvals.at[hvec].get(mode="promise_in_bounds")
            cid = hu.at[hvec].get(mode="promise_in_bounds") - base16
            for c in range(DC):
                g = plsc.load_gather(width_ref, [iota + c * L, cid])
                buf_ref[h, pl.ds(c * L, L)] = (g + tb * w_chunks[c]
                                               + bias_chunks[c])
            return carry
        lax.fori_loop(0, p2, hbody, 0)

    def drain(buf_ref, idx_ref, sem):
        pltpu.make_async_copy(buf_ref, out_hbm.at[idx_ref], sem).wait()

    def fire(buf_ref, idx_ref, sem):
        pltpu.async_copy(buf_ref, out_hbm.at[idx_ref], sem)

    idxa_v[pl.ds(0, L)] = trash16
    idxb_v[pl.ds(0, L)] = trash16
    fire(bufa_v, idxa_v, sema)
    fire(bufb_v, idxb_v, semb)

    def process(n_hits, width_ref, base_col):
        nch2 = (n_hits + 2 * L - 1) // (2 * L)

        def body2(k2, carry):
            drain(bufa_v, idxa_v, sema)
            build_chunk(2 * k2, n_hits, width_ref, base_col,
                        bufa_v, idxa_v)
            fire(bufa_v, idxa_v, sema)
            drain(bufb_v, idxb_v, semb)
            build_chunk(2 * k2 + 1, n_hits, width_ref, base_col,
                        bufb_v, idxb_v)
            fire(bufb_v, idxb_v, semb)
            return carry
        lax.fori_loop(0, nch2, body2, 0)

    for s, slo, shi in subs:
        ns = recompact(slo, shi)
        pltpu.make_async_copy(
            table_hbm.at[:, pl.ds(slab_src(slo), SLAB_W)],
            slabs[s % 2], slab_sems[s % 2]).wait()
        process(ns, slabs[s % 2], slab_src(slo))
        if s + 2 < N_SUB:
            nxt = subs[s + 2]
            pltpu.async_copy(
                table_hbm.at[:, pl.ds(slab_src(nxt[1]), SLAB_W)],
                slabs[s % 2], slab_sems[s % 2])

    nt = recompact(jnp.int32(UT_FULL), jnp.int32(UT_FULL + 2))
    pltpu.make_async_copy(tail_hbm, tail_v, semt).wait()
    process(nt, tail_v, jnp.int32(UT_TAIL_BASE))

    drain(bufa_v, idxa_v, sema)
    drain(bufb_v, idxb_v, semb)


@jax.jit
def _run(users, timestamps, table_t, tail, w_flat, time_b):
    mesh = plsc.VectorSubcoreMesh(core_axis_name="c", subcore_axis_name="s",
                                  num_cores=NC)
    return pl.kernel(
        _tae_kernel,
        out_type=jax.ShapeDtypeStruct((OUT_ROWS, 128), jnp.float32),
        mesh=mesh,
        compiler_params=pltpu.CompilerParams(needs_layout_passes=False),
        scratch_types=[
            pltpu.VMEM((BATCH,), jnp.int32),
            pltpu.VMEM((BATCH,), jnp.float32),
            pltpu.VMEM((EMBED_DIM, SLAB_W), jnp.float32),
            pltpu.VMEM((EMBED_DIM, SLAB_W), jnp.float32),
            pltpu.VMEM((EMBED_DIM, TAIL_N), jnp.float32),
            pltpu.VMEM((BATCH + L,), jnp.int32),
            pltpu.VMEM((BATCH + L,), jnp.int32),
            pltpu.VMEM((L, 128), jnp.float32),
            pltpu.VMEM((L, 128), jnp.float32),
            pltpu.VMEM((L,), jnp.int32),
            pltpu.VMEM((L,), jnp.int32),
            pltpu.VMEM((EMBED_DIM,), jnp.float32),
            pltpu.VMEM((EMBED_DIM,), jnp.float32),
            pltpu.SemaphoreType.DMA,
            pltpu.SemaphoreType.DMA,
            pltpu.SemaphoreType.DMA,
            pltpu.SemaphoreType.DMA,
            pltpu.SemaphoreType.DMA,
        ],
    )(users, timestamps, table_t, tail, w_flat, time_b)


def kernel(users, timestamps, table, time_w, time_b):
    table_t = table.T
    tail = table_t[:, UT_TAIL_BASE:]
    out1 = _run(users.astype(jnp.int32), timestamps, table_t, tail,
                time_w.reshape(EMBED_DIM), time_b)
    return out1[:BATCH, :EMBED_DIM]
